# baseline (speedup 1.0000x reference)
import functools

import jax
import jax.numpy as jnp
from jax import lax
from jax.experimental import pallas as pl
from jax.experimental.pallas import tpu as pltpu
from jax.experimental.pallas import tpu_sc as plsc

NUM_ROWS = 1_000_000
DIM = 32
B = 16384 * 50
IDX_COLS = 128
IDX_ROWS = B // IDX_COLS
NC, NS = 2, 16
NW = NC * NS
RPW = IDX_ROWS // NW
BPW = B // NW
CHUNK = IDX_COLS
NBUF = 4
GROUPS = RPW // NBUF


def _make_gather():
    mesh = plsc.VectorSubcoreMesh(
        core_axis_name="c", subcore_axis_name="s",
        num_cores=NC, num_subcores=NS)

    @functools.partial(
        pl.kernel,
        out_type=jax.ShapeDtypeStruct((B, DIM), jnp.float32),
        mesh=mesh,
        compiler_params=pltpu.CompilerParams(use_tc_tiling_on_sc=False),
        scratch_types=[
            pltpu.VMEM((RPW, IDX_COLS), jnp.int32),
            [pltpu.VMEM((CHUNK, DIM), jnp.float32) for _ in range(NBUF)],
            [pltpu.SemaphoreType.DMA for _ in range(NBUF)],
        ],
    )
    def gather_kernel(idx_hbm, table_hbm, out_hbm, idx_v, rows, gsems):
        wid = lax.axis_index("s") * NC + lax.axis_index("c")
        base = wid * BPW
        pltpu.sync_copy(idx_hbm.at[pl.ds(wid * RPW, RPW)], idx_v)

        def gather_chunk(j, b):
            src = table_hbm.at[idx_v.at[j]]
            return pltpu.make_async_copy(src, rows[b], gsems[b])

        for b in range(NBUF):
            gather_chunk(b, b).start()

        def group_body(g, _):
            j0 = g * NBUF
            for b in range(NBUF):
                j = j0 + b
                gather_chunk(j, b).wait()
                pltpu.sync_copy(
                    rows[b], out_hbm.at[pl.ds(base + j * CHUNK, CHUNK)])
                nxt = j + NBUF

                @pl.when(nxt < RPW)
                def _():
                    gather_chunk(nxt, b).start()
            return ()

        lax.fori_loop(0, GROUPS, group_body, (), unroll=False)

    return gather_kernel


_gather = _make_gather()


@jax.jit
def kernel(input, weight):
    idx = input.reshape(IDX_ROWS, IDX_COLS).astype(jnp.int32)
    out = _gather(idx, weight)
    return out.reshape(input.shape + (DIM,))


# speedup vs baseline: 1.6153x; 1.6153x over previous
import functools

import jax
import jax.numpy as jnp
from jax import lax
from jax.experimental import pallas as pl
from jax.experimental.pallas import tpu as pltpu
from jax.experimental.pallas import tpu_sc as plsc

NUM_ROWS = 1_000_000
DIM = 32
ROWS, COLS = 16384, 50
NC, NS = 2, 16
NW = NC * NS
RPW = ROWS // NW
NBUF = 8
STEPS = RPW
GROUPS = STEPS // NBUF


def _make_gather():
    mesh = plsc.VectorSubcoreMesh(
        core_axis_name="c", subcore_axis_name="s",
        num_cores=NC, num_subcores=NS)

    @functools.partial(
        pl.kernel,
        out_type=jax.ShapeDtypeStruct((ROWS, COLS, DIM), jnp.float32),
        mesh=mesh,
        compiler_params=pltpu.CompilerParams(use_tc_tiling_on_sc=False),
        scratch_types=[
            pltpu.VMEM((RPW, COLS), jnp.int32),
            [pltpu.VMEM((COLS, DIM), jnp.float32) for _ in range(NBUF)],
            [pltpu.SemaphoreType.DMA for _ in range(NBUF)],
        ],
    )
    def gather_kernel(idx_hbm, table_hbm, out_hbm, idx_v, rows, gsems):
        wid = lax.axis_index("s") * NC + lax.axis_index("c")
        base = wid * RPW
        pltpu.sync_copy(idx_hbm.at[pl.ds(base, RPW)], idx_v)

        def gather_chunk(j, b):
            src = table_hbm.at[idx_v.at[j]]
            return pltpu.make_async_copy(src, rows[b], gsems[b])

        for b in range(NBUF):
            gather_chunk(b, b).start()

        def group_body(g, _):
            j0 = g * NBUF
            for b in range(NBUF):
                j = j0 + b
                gather_chunk(j, b).wait()
                pltpu.sync_copy(rows[b], out_hbm.at[base + j])
                nxt = j + NBUF

                @pl.when(nxt < STEPS)
                def _():
                    gather_chunk(nxt, b).start()
            return ()

        lax.fori_loop(0, GROUPS, group_body, (), unroll=False)

    return gather_kernel


_gather = _make_gather()


@jax.jit
def kernel(input, weight):
    return _gather(input, weight)


# speedup vs baseline: 1.6160x; 1.0005x over previous
import functools

import jax
import jax.numpy as jnp
from jax import lax
from jax.experimental import pallas as pl
from jax.experimental.layout import Format, Layout
from jax.experimental.pallas import tpu as pltpu
from jax.experimental.pallas import tpu_sc as plsc

NUM_ROWS = 1_000_000
DIM = 32
ROWS, COLS = 16384, 50
COLS_PAD = 56
NC, NS = 2, 16
NW = NC * NS
RPW = ROWS // NW
NBUF = 8
STEPS = RPW
GROUPS = STEPS // NBUF


def _make_gather():
    mesh = plsc.VectorSubcoreMesh(
        core_axis_name="c", subcore_axis_name="s",
        num_cores=NC, num_subcores=NS)

    @functools.partial(
        pl.kernel,
        out_type=jax.ShapeDtypeStruct((ROWS, COLS_PAD, DIM), jnp.float32),
        mesh=mesh,
        compiler_params=pltpu.CompilerParams(use_tc_tiling_on_sc=False),
        scratch_types=[
            pltpu.VMEM((RPW, COLS), jnp.int32),
            [pltpu.VMEM((COLS, DIM), jnp.float32) for _ in range(NBUF)],
            [pltpu.SemaphoreType.DMA for _ in range(NBUF)],
        ],
    )
    def gather_kernel(idx_hbm, table_hbm, out_hbm, idx_v, rows, gsems):
        wid = lax.axis_index("s") * NC + lax.axis_index("c")
        base = wid * RPW
        pltpu.sync_copy(idx_hbm.at[pl.ds(base, RPW)], idx_v)

        def gather_chunk(j, b):
            src = table_hbm.at[idx_v.at[j]]
            return pltpu.make_async_copy(src, rows[b], gsems[b])

        for b in range(NBUF):
            gather_chunk(b, b).start()

        def group_body(g, _):
            j0 = g * NBUF
            for b in range(NBUF):
                j = j0 + b
                gather_chunk(j, b).wait()
                pltpu.sync_copy(
                    rows[b], out_hbm.at[base + j].at[pl.ds(0, COLS)])
                nxt = j + NBUF

                @pl.when(nxt < STEPS)
                def _():
                    gather_chunk(nxt, b).start()
            return ()

        lax.fori_loop(0, GROUPS, group_body, (), unroll=False)

    return gather_kernel


_gather = _make_gather()


def _kernel_impl(input, weight):
    out = _gather(input, weight)
    return lax.slice(out, (0, 0, 0), (ROWS, COLS, DIM))


_jitted = None


def kernel(input, weight):
    global _jitted
    if _jitted is None:
        sharding = jax.sharding.SingleDeviceSharding(jax.devices()[0])
        out_fmt = Format(Layout(major_to_minor=(0, 1, 2)), sharding)
        _jitted = jax.jit(_kernel_impl, out_shardings=out_fmt)
    return _jitted(input, weight)


# speedup vs baseline: 1.6399x; 1.0148x over previous
import functools

import jax
import jax.numpy as jnp
from jax import lax
from jax.experimental import pallas as pl
from jax.experimental.pallas import tpu as pltpu
from jax.experimental.pallas import tpu_sc as plsc

NUM_ROWS = 1_000_000
DIM = 32
ROWS, COLS = 16384, 50
COLS_PAD = 56
NC, NS = 2, 16
NW = NC * NS
IPW = ROWS // NW
CHUNK = 128
KPC = IPW // CHUNK
NBUF = 8
STEPS = COLS * KPC
GROUPS = STEPS // NBUF


def _make_gather():
    mesh = plsc.VectorSubcoreMesh(
        core_axis_name="c", subcore_axis_name="s",
        num_cores=NC, num_subcores=NS)

    @functools.partial(
        pl.kernel,
        out_type=jax.ShapeDtypeStruct((ROWS, COLS_PAD, DIM), jnp.float32),
        mesh=mesh,
        compiler_params=pltpu.CompilerParams(use_tc_tiling_on_sc=False),
        scratch_types=[
            pltpu.VMEM((COLS, IPW), jnp.int32),
            [pltpu.VMEM((CHUNK, DIM), jnp.float32) for _ in range(NBUF)],
            [pltpu.SemaphoreType.DMA for _ in range(NBUF)],
        ],
    )
    def gather_kernel(idx_hbm, table_hbm, out_hbm, idx_v, rows, gsems):
        wid = lax.axis_index("s") * NC + lax.axis_index("c")
        base = wid * IPW
        pltpu.sync_copy(idx_hbm.at[:, pl.ds(base, IPW)], idx_v)

        def gather_chunk(j, b):
            c = j // KPC
            k = j - c * KPC
            src = table_hbm.at[idx_v.at[c, pl.ds(k * CHUNK, CHUNK)]]
            return pltpu.make_async_copy(src, rows[b], gsems[b])

        for b in range(NBUF):
            gather_chunk(b, b).start()

        def group_body(g, _):
            j0 = g * NBUF
            for b in range(NBUF):
                j = j0 + b
                c = j // KPC
                k = j - c * KPC
                gather_chunk(j, b).wait()
                pltpu.sync_copy(
                    rows[b],
                    out_hbm.at[pl.ds(base + k * CHUNK, CHUNK), c])
                nxt = j + NBUF

                @pl.when(nxt < STEPS)
                def _():
                    gather_chunk(nxt, b).start()
            return ()

        lax.fori_loop(0, GROUPS, group_body, (), unroll=False)

    return gather_kernel


_gather = _make_gather()


@jax.jit
def kernel(input, weight):
    out = _gather(input.T, weight)
    return lax.slice(out, (0, 0, 0), (ROWS, COLS, DIM))
